# Initial kernel scaffold; baseline (speedup 1.0000x reference)
import jax
import jax.numpy as jnp
from jax.experimental import pallas as pl


def kernel(Z, R, e_source, e_target, molecule_ind, total_charge, params):
    raise NotImplementedError("write your pallas kernel here")


# speedup vs baseline: 86.3669x; 86.3669x over previous
import functools

import jax
import jax.numpy as jnp
from jax import lax
from jax.experimental import pallas as pl
from jax.experimental.pallas import tpu as pltpu
from jax.experimental.pallas import tpu_sc as plsc

N_RBF = 8
N_EMBED = 8
N_MSG = 3
R_CUT = 5.0
MAX_Z = 118
N_ATOM = 10000
N_EDGE = 320000
N_MOL = 100

NC, NS, L = 2, 16, 16
NW = NC * NS
NPAD = 10240
EW = N_EDGE // NW
K = 400
GW = 18
SW = 96
DW = 80
BN = 512

_i32 = jnp.int32
_f32 = jnp.float32


def _iota16():
    return lax.iota(_i32, L)


def _splat(c):
    return jnp.full((L,), c, _i32)


def _sqrt16(x):
    i = plsc.bitcast(x, _i32)
    i = 0x1FBD1DF5 + lax.shift_right_logical(i, 1)
    y = plsc.bitcast(i, _f32)
    for _ in range(3):
        y = 0.5 * (y + x / y)
    return y


def _embed_body(z_hbm, emb_hbm, gus_hbm, h0_hbm, q0_hbm, z_v, emb_v, gus_v,
                h0_v, q0_v):
    cid = lax.axis_index("c")
    sid = lax.axis_index("s")
    wid = cid * NS + sid
    npw = NPAD // NW
    base = wid * npw
    pltpu.sync_copy(z_hbm.at[pl.ds(base, npw)], z_v)
    pltpu.sync_copy(emb_hbm, emb_v)
    pltpu.sync_copy(gus_hbm, gus_v)

    def grp(j, carry):
        pos = j * L + _iota16()
        zv = z_v[pl.ds(j * L, L)]
        for z in range(N_EMBED):
            hv = plsc.load_gather(emb_v, [zv, _splat(z)])
            plsc.store_scatter(h0_v, [pos, _splat(z)], hv)
        qv = plsc.load_gather(gus_v, [zv, _splat(0)])
        q0_v[pl.ds(j * L, L)] = qv
        return carry

    lax.fori_loop(0, npw // L, grp, 0)
    pltpu.sync_copy(h0_v, h0_hbm.at[pl.ds(base, npw)])
    pltpu.sync_copy(q0_v, q0_hbm.at[pl.ds(base, npw)])


@jax.jit
def _embed_call(z_pad, emb, gus):
    npw = NPAD // NW
    mesh = plsc.VectorSubcoreMesh(core_axis_name="c", subcore_axis_name="s")
    return pl.kernel(
        _embed_body,
        out_type=(jax.ShapeDtypeStruct((NPAD, N_EMBED), _f32),
                  jax.ShapeDtypeStruct((NPAD,), _f32)),
        mesh=mesh,
        compiler_params=pltpu.CompilerParams(needs_layout_passes=False, use_tc_tiling_on_sc=False),
        scratch_types=[
            pltpu.VMEM((npw,), _i32),
            pltpu.VMEM((MAX_Z + 1, N_EMBED), _f32),
            pltpu.VMEM((MAX_Z + 1, 1), _f32),
            pltpu.VMEM((npw, N_EMBED), _f32),
            pltpu.VMEM((npw,), _f32),
        ],
    )(z_pad, emb, gus)


def _geom_body(r_hbm, esrc_hbm, etgt_hbm, geom_hbm, r_v, isrc_v, itgt_v, g_v):
    cid = lax.axis_index("c")
    sid = lax.axis_index("s")
    wid = cid * NS + sid
    ebase = wid * EW
    pltpu.sync_copy(r_hbm, r_v)
    mus = [R_CUT * r / (N_RBF - 1) for r in range(N_RBF)]
    inv_g2 = (N_RBF / R_CUT) ** 2

    def chunk(ch, carry):
        gb = ebase + ch * K
        pltpu.sync_copy(esrc_hbm.at[pl.ds(gb, K)], isrc_v)
        pltpu.sync_copy(etgt_hbm.at[pl.ds(gb, K)], itgt_v)

        def grp(j, c2):
            pos = j * L + _iota16()
            rs = isrc_v[pl.ds(j * L, L)]
            rt = itgt_v[pl.ds(j * L, L)]
            d = []
            for c in range(3):
                xs = plsc.load_gather(r_v, [rs, _splat(c)])
                xt = plsc.load_gather(r_v, [rt, _splat(c)])
                d.append(xt - xs)
            d2 = d[0] * d[0] + d[1] * d[1] + d[2] * d[2]
            dr = _sqrt16(jnp.maximum(d2, 1e-12))
            inv = 1.0 / jnp.maximum(dr, 1e-6)
            dru = [dc * inv for dc in d]
            plsc.store_scatter(g_v, [pos, _splat(0)], jnp.full((L,), 1.0, _f32))
            for r in range(N_RBF):
                t = dr - mus[r]
                plsc.store_scatter(g_v, [pos, _splat(1 + r)],
                                   jnp.exp(-(t * t) * inv_g2))
            for c in range(3):
                plsc.store_scatter(g_v, [pos, _splat(9 + c)], dru[c])
            o6 = [dru[0] * dru[0], dru[0] * dru[1], dru[0] * dru[2],
                  dru[1] * dru[1], dru[1] * dru[2], dru[2] * dru[2]]
            for c in range(6):
                plsc.store_scatter(g_v, [pos, _splat(12 + c)], o6[c])
            return c2

        lax.fori_loop(0, K // L, grp, 0)
        pltpu.sync_copy(g_v, geom_hbm.at[pl.ds(gb, K), :])
        return carry

    lax.fori_loop(0, EW // K, chunk, 0)


@jax.jit
def _geom_call(r, esrc, etgt):
    mesh = plsc.VectorSubcoreMesh(core_axis_name="c", subcore_axis_name="s")
    return pl.kernel(
        _geom_body,
        out_type=jax.ShapeDtypeStruct((N_EDGE, GW), _f32),
        mesh=mesh,
        compiler_params=pltpu.CompilerParams(needs_layout_passes=False, use_tc_tiling_on_sc=False),
        scratch_types=[
            pltpu.VMEM((N_ATOM, 3), _f32),
            pltpu.VMEM((K,), _i32),
            pltpu.VMEM((K,), _i32),
            pltpu.VMEM((K, GW), _f32),
        ],
    )(r, esrc, etgt)


def _edge_body(static, c0, W, tab_hbm, geom_hbm, esrc_hbm, etgt_hbm, out_hbm,
               rows_v, stage_v, g_v, isrc_v, itgt_v, sem, acc_sh):
    cid = lax.axis_index("c")
    sid = lax.axis_index("s")
    wid = cid * NS + sid
    ebase = wid * EW

    def zr(r, c2):
        for c in range(W // L):
            plsc.store_scatter(stage_v, [jnp.full((L,), r, _i32),
                                         c * L + _iota16()],
                               jnp.zeros((L,), _f32))
        return c2

    lax.fori_loop(0, K, zr, 0)
    nzr = NPAD // NS
    pltpu.sync_copy(stage_v, acc_sh.at[pl.ds(sid * nzr, K)])
    pltpu.sync_copy(stage_v.at[pl.ds(0, nzr - K)],
                    acc_sh.at[pl.ds(sid * nzr + K, nzr - K)])
    plsc.subcore_barrier()

    def chunk(ch, carry):
        gb = ebase + ch * K
        pltpu.sync_copy(esrc_hbm.at[pl.ds(gb, K)], isrc_v)
        pltpu.sync_copy(etgt_hbm.at[pl.ds(gb, K)], itgt_v)
        pltpu.sync_copy(geom_hbm.at[pl.ds(gb, K), :], g_v)
        pltpu.async_copy(tab_hbm.at[itgt_v], rows_v, sem).wait()

        def grp(j, c2):
            pos = j * L + _iota16()
            hz = [plsc.load_gather(rows_v, [pos, _splat(z)])
                  for z in range(N_EMBED)]
            rb = [plsc.load_gather(g_v, [pos, _splat(1 + r)])
                  for r in range(N_RBF)]
            feats = []
            if static:
                feats.append((0, lambda: plsc.load_gather(
                    g_v, [pos, _splat(0)])))
                for r in range(N_RBF):
                    feats.append((1 + r, lambda r=r: rb[r]))
                off = 9
                for c in range(9):
                    feats.append((81 + c, lambda c=c: plsc.load_gather(
                        g_v, [pos, _splat(9 + c)])))
            else:
                off = 0
            for z in range(N_EMBED):
                feats.append((off + z, lambda z=z: hz[z]))
            for z in range(N_EMBED):
                for r in range(N_RBF):
                    feats.append((off + N_EMBED + z * N_RBF + r,
                                  lambda z=z, r=r: hz[z] * rb[r]))
            for gc, tk in feats:
                if c0 <= gc < c0 + W:
                    plsc.store_scatter(stage_v, [pos, _splat(gc - c0)], tk())
            return c2

        lax.fori_loop(0, K // L, grp, 0)
        pltpu.sync_copy(stage_v, acc_sh.at[isrc_v], add=True)
        return carry

    lax.fori_loop(0, EW // K, chunk, 0)
    plsc.subcore_barrier()
    nzc = NPAD // NS
    pltpu.sync_copy(acc_sh.at[pl.ds(sid * nzc, nzc)],
                    out_hbm.at[cid, pl.ds(sid * nzc, nzc), :])


@functools.partial(jax.jit, static_argnums=(0, 1, 2))
def _edge_call(static, c0, W, tab, geom, esrc, etgt):
    mesh = plsc.VectorSubcoreMesh(core_axis_name="c", subcore_axis_name="s")
    return pl.kernel(
        functools.partial(_edge_body, static, c0, W),
        out_type=jax.ShapeDtypeStruct((NC, NPAD, W), _f32),
        mesh=mesh,
        compiler_params=pltpu.CompilerParams(needs_layout_passes=False, use_tc_tiling_on_sc=False),
        scratch_types=[
            pltpu.VMEM((K, N_EMBED), _f32),
            pltpu.VMEM((K, W), _f32),
            pltpu.VMEM((K, GW), _f32),
            pltpu.VMEM((K,), _i32),
            pltpu.VMEM((K,), _i32),
            pltpu.SemaphoreType.DMA,
            pltpu.VMEM_SHARED((NPAD, W), _f32),
        ],
    )(tab, geom, esrc, etgt)


def _ffn_body(h0_ref, h_ref, sa_ref, sb_ref, ta_ref, tb_ref, q_ref, d_ref,
              p_ref, w1_ref, b1_ref, w2a, w2b, w2c, w2d, w2e, b2_ref,
              w3a, w3b, w3c, w3d, w3e, b3_ref, wh_ref, bh_ref, wc_ref, bc_ref,
              wd_ref, bd_ref, wq_ref, bq_ref,
              hn_ref, qo_ref, do_ref, po_ref):
    h0 = h0_ref[...]
    h = h_ref[...]
    S = sa_ref[...] + sb_ref[...]
    T = ta_ref[...] + tb_ref[...]
    deg = S[:, 0:1]
    rbf_sum = S[:, 1:9]
    s0 = S[:, 9:17]
    T0 = S[:, 17:81]
    dru_sum = S[:, 81:84]
    o6 = S[:, 84:90]
    sh = T[:, 0:8]
    Th = T[:, 8:72]
    out0 = [h0[:, z:z + 1] * rbf_sum for z in range(N_EMBED)]
    outh = [h[:, z:z + 1] * rbf_sum for z in range(N_EMBED)]
    m = jnp.concatenate(
        [h0 * deg, s0, h * deg, sh] + out0 + [T0] + outh + [Th, rbf_sum],
        axis=1)
    h1 = jax.nn.relu(jnp.dot(m, w1_ref[...],
                             preferred_element_type=_f32) + b1_ref[...])
    w2 = (w2a, w2b, w2c, w2d, w2e)
    w3 = (w3a, w3b, w3c, w3d, w3e)
    h3 = []
    for f in range(5):
        h2f = jax.nn.relu(
            jnp.dot(h1[:, f * 256:(f + 1) * 256], w2[f][...],
                    preferred_element_type=_f32)
            + b2_ref[:, f * 128:(f + 1) * 128])
        h3f = jax.nn.relu(
            jnp.dot(h2f, w3[f][...], preferred_element_type=_f32)
            + b3_ref[:, f * 64:(f + 1) * 64])
        h3.append(h3f)
    hn_ref[...] = jnp.dot(h3[0], wh_ref[...],
                          preferred_element_type=_f32) + bh_ref[...]
    dq = jnp.dot(h3[1], wc_ref[...], preferred_element_type=_f32) + bc_ref[...]
    qo_ref[...] = q_ref[...] + dq
    sd = jnp.dot(h3[2], wd_ref[...], preferred_element_type=_f32) + bd_ref[...]
    do_ref[...] = d_ref[...] + sd * dru_sum
    sq1 = jnp.dot(h3[3], wq_ref[...], preferred_element_type=_f32) + bq_ref[...]
    sq2 = jnp.dot(h3[4], wq_ref[...], preferred_element_type=_f32) + bq_ref[...]
    o9 = jnp.concatenate(
        [o6[:, 0:3], o6[:, 1:2], o6[:, 3:5], o6[:, 2:3], o6[:, 4:6]], axis=1)
    eye9 = (lax.broadcasted_iota(_i32, (1, 9), 1) % 4 == 0).astype(_f32)
    po_ref[...] = p_ref[...] + sq1 * o9 + (sq2 * deg) * eye9


@jax.jit
def _ffn_call(h0, h, sa, sb, ta, tb, q, d, p, weights):
    grid = (NPAD // BN,)

    def nspec(c):
        return pl.BlockSpec((BN, c), lambda i: (i, 0))

    def fspec(shape):
        nd = len(shape)
        return pl.BlockSpec(shape, lambda i: (0,) * nd)

    in_specs = ([nspec(8), nspec(8), nspec(SW), nspec(SW), nspec(DW),
                 nspec(DW), nspec(1), nspec(3), nspec(9)]
                + [fspec(w.shape) for w in weights])
    out_specs = [nspec(8), nspec(1), nspec(3), nspec(9)]
    out_shape = [jax.ShapeDtypeStruct((NPAD, 8), _f32),
                 jax.ShapeDtypeStruct((NPAD, 1), _f32),
                 jax.ShapeDtypeStruct((NPAD, 3), _f32),
                 jax.ShapeDtypeStruct((NPAD, 9), _f32)]
    return pl.pallas_call(
        _ffn_body, grid=grid, in_specs=in_specs, out_specs=out_specs,
        out_shape=out_shape,
    )(h0, h, sa, sb, ta, tb, q, d, p, *weights)


def _mol_reduce_body(q_ref, ind_ref, out_ref):
    i = pl.program_id(0)

    @pl.when(i == 0)
    def _():
        out_ref[...] = jnp.zeros_like(out_ref)

    ind = ind_ref[0, 0, :]
    M = (ind[:, None] == lax.broadcasted_iota(_i32, (BN, 128), 1)).astype(_f32)
    q = q_ref[...]
    molq = jnp.sum(M * q, axis=0, keepdims=True)
    nper = jnp.sum(M, axis=0, keepdims=True)
    out_ref[...] = out_ref[...] + jnp.concatenate([molq, nper], axis=0)


def _mol_apply_body(q_ref, ind_ref, st_ref, tq_ref, out_ref):
    molq = st_ref[0:1, :]
    nper = jnp.maximum(st_ref[1:2, :], 1.0)
    corr = (tq_ref[...] - molq) / nper
    ind = ind_ref[0, 0, :]
    M = (ind[:, None] == lax.broadcasted_iota(_i32, (BN, 128), 1)).astype(_f32)
    out_ref[...] = q_ref[...] + jnp.sum(M * corr, axis=1, keepdims=True)


@jax.jit
def _mol_call(q, ind3, tq):
    grid = (NPAD // BN,)
    nspec = pl.BlockSpec((BN, 1), lambda i: (i, 0))
    ispec = pl.BlockSpec((1, 1, BN), lambda i: (i, 0, 0))
    sspec = pl.BlockSpec((2, 128), lambda i: (0, 0))
    st = pl.pallas_call(
        _mol_reduce_body, grid=grid, in_specs=[nspec, ispec],
        out_specs=sspec,
        out_shape=jax.ShapeDtypeStruct((2, 128), _f32),
    )(q, ind3)
    tspec = pl.BlockSpec((1, 128), lambda i: (0, 0))
    return pl.pallas_call(
        _mol_apply_body, grid=grid,
        in_specs=[nspec, ispec, sspec, tspec], out_specs=nspec,
        out_shape=jax.ShapeDtypeStruct((NPAD, 1), _f32),
    )(q, ind3, st, tq)


def kernel(Z, R, e_source, e_target, molecule_ind, total_charge, params):
    Z = Z.astype(_i32)
    esrc = e_source.astype(_i32)
    etgt = e_target.astype(_i32)
    z_pad = jnp.concatenate([Z, jnp.zeros((NPAD - N_ATOM,), _i32)])
    h0, q0 = _embed_call(z_pad, params["embed"], params["guess"])
    geom = _geom_call(R.astype(_f32), esrc, etgt)
    S = _edge_call(True, 0, SW, h0, geom, esrc, etgt)

    charge = q0.reshape(NPAD, 1)
    dipole = jnp.zeros((NPAD, 3), _f32)
    qpole = jnp.zeros((NPAD, 9), _f32)
    h = h0
    for i in range(N_MSG):
        T = _edge_call(False, 0, DW, h, geom, esrc, etgt)
        ps = params
        w1 = jnp.concatenate(
            [ps[n % i][0][0] for n in ("charge_update_%d", "charge_readout_%d",
                                       "dipole_update_%d", "qpole1_update_%d",
                                       "qpole2_update_%d")], axis=1)
        b1 = jnp.concatenate(
            [ps[n % i][0][1] for n in ("charge_update_%d", "charge_readout_%d",
                                       "dipole_update_%d", "qpole1_update_%d",
                                       "qpole2_update_%d")]).reshape(1, -1)
        names = ("charge_update_%d", "charge_readout_%d", "dipole_update_%d",
                 "qpole1_update_%d", "qpole2_update_%d")
        w2s = [ps[n % i][1][0] for n in names]
        b2 = jnp.concatenate([ps[n % i][1][1] for n in names]).reshape(1, -1)
        w3s = [ps[n % i][2][0] for n in names]
        b3 = jnp.concatenate([ps[n % i][2][1] for n in names]).reshape(1, -1)
        wh, bh = ps["charge_update_%d" % i][3]
        wc, bc = ps["charge_readout_%d" % i][3]
        wd, bd = ps["dipole_readout_%d" % i]
        wq, bq = ps["qpole_readout_%d" % i]
        weights = ([w1, b1] + w2s + [b2] + w3s
                   + [b3, wh, bh.reshape(1, -1), wc, bc.reshape(1, -1),
                      wd, bd.reshape(1, -1), wq, bq.reshape(1, -1)])
        h, charge, dipole, qpole = _ffn_call(
            h0, h, S[0], S[1], T[0], T[1], charge, dipole, qpole, weights)

    ind3 = jnp.concatenate(
        [molecule_ind.astype(_i32),
         jnp.full((NPAD - N_ATOM,), 127, _i32)]).reshape(NPAD // BN, 1, BN)
    tq = jnp.concatenate([total_charge.astype(_f32),
                          jnp.zeros((128 - N_MOL,), _f32)]).reshape(1, 128)
    charge = _mol_call(charge, ind3, tq)

    return (charge[:N_ATOM],
            dipole[:N_ATOM],
            qpole[:N_ATOM].reshape(N_ATOM, 3, 3))


# speedup vs baseline: 103.0175x; 1.1928x over previous
import functools

import jax
import jax.numpy as jnp
from jax import lax
from jax.experimental import pallas as pl
from jax.experimental.pallas import tpu as pltpu
from jax.experimental.pallas import tpu_sc as plsc

N_RBF = 8
N_EMBED = 8
N_MSG = 3
R_CUT = 5.0
MAX_Z = 118
N_ATOM = 10000
N_EDGE = 320000
N_MOL = 100

NC, NS, L = 2, 16, 16
NW = NC * NS
NPAD = 10240
EW = N_EDGE // NW
K = 400
GW = 18
SW = 96
DW = 80
BN = 512

_i32 = jnp.int32
_f32 = jnp.float32


def _iota16():
    return lax.iota(_i32, L)


def _splat(c):
    return jnp.full((L,), c, _i32)


def _sqrt16(x):
    i = plsc.bitcast(x, _i32)
    i = 0x1FBD1DF5 + lax.shift_right_logical(i, 1)
    y = plsc.bitcast(i, _f32)
    for _ in range(3):
        y = 0.5 * (y + x / y)
    return y


def _embed_body(z_hbm, emb_hbm, gus_hbm, h0_hbm, q0_hbm, z_v, emb_v, gus_v,
                h0_v, q0_v):
    cid = lax.axis_index("c")
    sid = lax.axis_index("s")
    wid = cid * NS + sid
    npw = NPAD // NW
    base = wid * npw
    pltpu.sync_copy(z_hbm.at[pl.ds(base, npw)], z_v)
    pltpu.sync_copy(emb_hbm, emb_v)
    pltpu.sync_copy(gus_hbm, gus_v)

    def grp(j, carry):
        pos = j * L + _iota16()
        zv = z_v[pl.ds(j * L, L)]
        for z in range(N_EMBED):
            hv = plsc.load_gather(emb_v, [zv, _splat(z)])
            plsc.store_scatter(h0_v, [pos, _splat(z)], hv)
        qv = plsc.load_gather(gus_v, [zv, _splat(0)])
        q0_v[pl.ds(j * L, L)] = qv
        return carry

    lax.fori_loop(0, npw // L, grp, 0)
    pltpu.sync_copy(h0_v, h0_hbm.at[pl.ds(base, npw)])
    pltpu.sync_copy(q0_v, q0_hbm.at[pl.ds(base, npw)])


@jax.jit
def _embed_call(z_pad, emb, gus):
    npw = NPAD // NW
    mesh = plsc.VectorSubcoreMesh(core_axis_name="c", subcore_axis_name="s")
    return pl.kernel(
        _embed_body,
        out_type=(jax.ShapeDtypeStruct((NPAD, N_EMBED), _f32),
                  jax.ShapeDtypeStruct((NPAD,), _f32)),
        mesh=mesh,
        compiler_params=pltpu.CompilerParams(needs_layout_passes=False, use_tc_tiling_on_sc=False),
        scratch_types=[
            pltpu.VMEM((npw,), _i32),
            pltpu.VMEM((MAX_Z + 1, N_EMBED), _f32),
            pltpu.VMEM((MAX_Z + 1, 1), _f32),
            pltpu.VMEM((npw, N_EMBED), _f32),
            pltpu.VMEM((npw,), _f32),
        ],
    )(z_pad, emb, gus)


def _geom_body(r_hbm, esrc_hbm, etgt_hbm, geom_hbm, r_v, isrc_v, itgt_v, g_v):
    cid = lax.axis_index("c")
    sid = lax.axis_index("s")
    wid = cid * NS + sid
    ebase = wid * EW
    pltpu.sync_copy(r_hbm, r_v)
    mus = [R_CUT * r / (N_RBF - 1) for r in range(N_RBF)]
    inv_g2 = (N_RBF / R_CUT) ** 2

    def chunk(ch, carry):
        gb = ebase + ch * K
        pltpu.sync_copy(esrc_hbm.at[pl.ds(gb, K)], isrc_v)
        pltpu.sync_copy(etgt_hbm.at[pl.ds(gb, K)], itgt_v)

        def grp(j, c2):
            pos = j * L + _iota16()
            rs = isrc_v[pl.ds(j * L, L)]
            rt = itgt_v[pl.ds(j * L, L)]
            d = []
            for c in range(3):
                xs = plsc.load_gather(r_v, [rs, _splat(c)])
                xt = plsc.load_gather(r_v, [rt, _splat(c)])
                d.append(xt - xs)
            d2 = d[0] * d[0] + d[1] * d[1] + d[2] * d[2]
            dr = _sqrt16(jnp.maximum(d2, 1e-12))
            inv = 1.0 / jnp.maximum(dr, 1e-6)
            dru = [dc * inv for dc in d]
            plsc.store_scatter(g_v, [pos, _splat(0)], jnp.full((L,), 1.0, _f32))
            for r in range(N_RBF):
                t = dr - mus[r]
                plsc.store_scatter(g_v, [pos, _splat(1 + r)],
                                   jnp.exp(-(t * t) * inv_g2))
            for c in range(3):
                plsc.store_scatter(g_v, [pos, _splat(9 + c)], dru[c])
            o6 = [dru[0] * dru[0], dru[0] * dru[1], dru[0] * dru[2],
                  dru[1] * dru[1], dru[1] * dru[2], dru[2] * dru[2]]
            for c in range(6):
                plsc.store_scatter(g_v, [pos, _splat(12 + c)], o6[c])
            return c2

        lax.fori_loop(0, K // L, grp, 0)
        pltpu.sync_copy(g_v, geom_hbm.at[pl.ds(gb, K), :])
        return carry

    lax.fori_loop(0, EW // K, chunk, 0)


@jax.jit
def _geom_call(r, esrc, etgt):
    mesh = plsc.VectorSubcoreMesh(core_axis_name="c", subcore_axis_name="s")
    return pl.kernel(
        _geom_body,
        out_type=jax.ShapeDtypeStruct((N_EDGE, GW), _f32),
        mesh=mesh,
        compiler_params=pltpu.CompilerParams(needs_layout_passes=False, use_tc_tiling_on_sc=False),
        scratch_types=[
            pltpu.VMEM((N_ATOM, 3), _f32),
            pltpu.VMEM((K,), _i32),
            pltpu.VMEM((K,), _i32),
            pltpu.VMEM((K, GW), _f32),
        ],
    )(r, esrc, etgt)


def _edge_body(static, c0, W, tab_hbm, geom_hbm, esrc_hbm, etgt_hbm, out_hbm,
               rows_v, stage_v, g_v, isrc_v, itgt_v, sem, acc_sh):
    cid = lax.axis_index("c")
    sid = lax.axis_index("s")
    wid = cid * NS + sid
    ebase = wid * EW

    def zr(r, c2):
        for c in range(W // L):
            plsc.store_scatter(stage_v, [jnp.full((L,), r, _i32),
                                         c * L + _iota16()],
                               jnp.zeros((L,), _f32))
        return c2

    lax.fori_loop(0, K, zr, 0)
    nzr = NPAD // NS
    pltpu.sync_copy(stage_v, acc_sh.at[pl.ds(sid * nzr, K)])
    pltpu.sync_copy(stage_v.at[pl.ds(0, nzr - K)],
                    acc_sh.at[pl.ds(sid * nzr + K, nzr - K)])
    plsc.subcore_barrier()

    def chunk(ch, carry):
        gb = ebase + ch * K
        pltpu.sync_copy(esrc_hbm.at[pl.ds(gb, K)], isrc_v)
        pltpu.sync_copy(etgt_hbm.at[pl.ds(gb, K)], itgt_v)
        pltpu.sync_copy(geom_hbm.at[pl.ds(gb, K), :], g_v)
        pltpu.async_copy(tab_hbm.at[itgt_v], rows_v, sem).wait()

        def grp(j, c2):
            pos = j * L + _iota16()
            hz = [plsc.load_gather(rows_v, [pos, _splat(z)])
                  for z in range(N_EMBED)]
            rb = [plsc.load_gather(g_v, [pos, _splat(1 + r)])
                  for r in range(N_RBF)]
            feats = []
            if static:
                feats.append((0, lambda: plsc.load_gather(
                    g_v, [pos, _splat(0)])))
                for r in range(N_RBF):
                    feats.append((1 + r, lambda r=r: rb[r]))
                off = 9
                for c in range(9):
                    feats.append((81 + c, lambda c=c: plsc.load_gather(
                        g_v, [pos, _splat(9 + c)])))
            else:
                off = 0
            for z in range(N_EMBED):
                feats.append((off + z, lambda z=z: hz[z]))
            for z in range(N_EMBED):
                for r in range(N_RBF):
                    feats.append((off + N_EMBED + z * N_RBF + r,
                                  lambda z=z, r=r: hz[z] * rb[r]))
            for gc, tk in feats:
                if c0 <= gc < c0 + W:
                    plsc.store_scatter(stage_v, [pos, _splat(gc - c0)], tk())
            return c2

        lax.fori_loop(0, K // L, grp, 0)
        pltpu.sync_copy(stage_v, acc_sh.at[isrc_v], add=True)
        return carry

    lax.fori_loop(0, EW // K, chunk, 0)
    plsc.subcore_barrier()
    nzc = NPAD // NS
    pltpu.sync_copy(acc_sh.at[pl.ds(sid * nzc, nzc)],
                    out_hbm.at[cid, pl.ds(sid * nzc, nzc), :])


@functools.partial(jax.jit, static_argnums=(0, 1, 2))
def _edge_call(static, c0, W, tab, geom, esrc, etgt):
    mesh = plsc.VectorSubcoreMesh(core_axis_name="c", subcore_axis_name="s")
    return pl.kernel(
        functools.partial(_edge_body, static, c0, W),
        out_type=jax.ShapeDtypeStruct((NC, NPAD, W), _f32),
        mesh=mesh,
        compiler_params=pltpu.CompilerParams(needs_layout_passes=False, use_tc_tiling_on_sc=False),
        scratch_types=[
            pltpu.VMEM((K, N_EMBED), _f32),
            pltpu.VMEM((K, W), _f32),
            pltpu.VMEM((K, GW), _f32),
            pltpu.VMEM((K,), _i32),
            pltpu.VMEM((K,), _i32),
            pltpu.SemaphoreType.DMA,
            pltpu.VMEM_SHARED((NPAD, W), _f32),
        ],
    )(tab, geom, esrc, etgt)


def _ffn_body(h0_ref, h_ref, sa_ref, sb_ref, ta_ref, tb_ref, q_ref, d_ref,
              p_ref, w1_ref, b1_ref, w2a, w2b, w2c, w2d, w2e, b2_ref,
              w3a, w3b, w3c, w3d, w3e, b3_ref, wh_ref, bh_ref, wc_ref, bc_ref,
              wd_ref, bd_ref, wq_ref, bq_ref,
              hn_ref, qo_ref, do_ref, po_ref):
    h0 = h0_ref[...]
    h = h_ref[...]
    S = sa_ref[...] + sb_ref[...]
    T = ta_ref[...] + tb_ref[...]
    deg = S[:, 0:1]
    rbf_sum = S[:, 1:9]
    s0 = S[:, 9:17]
    T0 = S[:, 17:81]
    dru_sum = S[:, 81:84]
    o6 = S[:, 84:90]
    sh = T[:, 0:8]
    Th = T[:, 8:72]
    out0 = [h0[:, z:z + 1] * rbf_sum for z in range(N_EMBED)]
    outh = [h[:, z:z + 1] * rbf_sum for z in range(N_EMBED)]
    m = jnp.concatenate(
        [h0 * deg, s0, h * deg, sh] + out0 + [T0] + outh + [Th, rbf_sum],
        axis=1)
    h1 = jax.nn.relu(jnp.dot(m, w1_ref[...],
                             preferred_element_type=_f32) + b1_ref[...])
    w2 = (w2a, w2b, w2c, w2d, w2e)
    w3 = (w3a, w3b, w3c, w3d, w3e)
    h3 = []
    for f in range(5):
        h2f = jax.nn.relu(
            jnp.dot(h1[:, f * 256:(f + 1) * 256], w2[f][...],
                    preferred_element_type=_f32)
            + b2_ref[:, f * 128:(f + 1) * 128])
        h3f = jax.nn.relu(
            jnp.dot(h2f, w3[f][...], preferred_element_type=_f32)
            + b3_ref[:, f * 64:(f + 1) * 64])
        h3.append(h3f)
    hn_ref[...] = jnp.dot(h3[0], wh_ref[...],
                          preferred_element_type=_f32) + bh_ref[...]
    dq = jnp.dot(h3[1], wc_ref[...], preferred_element_type=_f32) + bc_ref[...]
    qo_ref[...] = q_ref[...] + dq
    sd = jnp.dot(h3[2], wd_ref[...], preferred_element_type=_f32) + bd_ref[...]
    do_ref[...] = d_ref[...] + sd * dru_sum
    sq1 = jnp.dot(h3[3], wq_ref[...], preferred_element_type=_f32) + bq_ref[...]
    sq2 = jnp.dot(h3[4], wq_ref[...], preferred_element_type=_f32) + bq_ref[...]
    o9 = jnp.concatenate(
        [o6[:, 0:3], o6[:, 1:2], o6[:, 3:5], o6[:, 2:3], o6[:, 4:6]], axis=1)
    eye9 = (lax.broadcasted_iota(_i32, (1, 9), 1) % 4 == 0).astype(_f32)
    po_ref[...] = p_ref[...] + sq1 * o9 + (sq2 * deg) * eye9


@jax.jit
def _ffn_call(h0, h, sa, sb, ta, tb, q, d, p, weights):
    grid = (NPAD // BN,)

    def nspec(c):
        return pl.BlockSpec((BN, c), lambda i: (i, 0))

    def fspec(shape):
        nd = len(shape)
        return pl.BlockSpec(shape, lambda i: (0,) * nd)

    in_specs = ([nspec(8), nspec(8), nspec(SW), nspec(SW), nspec(DW),
                 nspec(DW), nspec(1), nspec(3), nspec(9)]
                + [fspec(w.shape) for w in weights])
    out_specs = [nspec(8), nspec(1), nspec(3), nspec(9)]
    out_shape = [jax.ShapeDtypeStruct((NPAD, 8), _f32),
                 jax.ShapeDtypeStruct((NPAD, 1), _f32),
                 jax.ShapeDtypeStruct((NPAD, 3), _f32),
                 jax.ShapeDtypeStruct((NPAD, 9), _f32)]
    return pl.pallas_call(
        _ffn_body, grid=grid, in_specs=in_specs, out_specs=out_specs,
        out_shape=out_shape,
    )(h0, h, sa, sb, ta, tb, q, d, p, *weights)


def _mol_reduce_body(q_ref, ind_ref, out_ref):
    i = pl.program_id(0)

    @pl.when(i == 0)
    def _():
        out_ref[...] = jnp.zeros_like(out_ref)

    ind = ind_ref[0, 0, :]
    M = (ind[:, None] == lax.broadcasted_iota(_i32, (BN, 128), 1)).astype(_f32)
    q = q_ref[...]
    molq = jnp.sum(M * q, axis=0, keepdims=True)
    nper = jnp.sum(M, axis=0, keepdims=True)
    out_ref[...] = out_ref[...] + jnp.concatenate([molq, nper], axis=0)


def _mol_apply_body(q_ref, ind_ref, st_ref, tq_ref, out_ref):
    molq = st_ref[0:1, :]
    nper = jnp.maximum(st_ref[1:2, :], 1.0)
    corr = (tq_ref[...] - molq) / nper
    ind = ind_ref[0, 0, :]
    M = (ind[:, None] == lax.broadcasted_iota(_i32, (BN, 128), 1)).astype(_f32)
    out_ref[...] = q_ref[...] + jnp.sum(M * corr, axis=1, keepdims=True)


@jax.jit
def _mol_call(q, ind3, tq):
    grid = (NPAD // BN,)
    nspec = pl.BlockSpec((BN, 1), lambda i: (i, 0))
    ispec = pl.BlockSpec((1, 1, BN), lambda i: (i, 0, 0))
    sspec = pl.BlockSpec((2, 128), lambda i: (0, 0))
    st = pl.pallas_call(
        _mol_reduce_body, grid=grid, in_specs=[nspec, ispec],
        out_specs=sspec,
        out_shape=jax.ShapeDtypeStruct((2, 128), _f32),
    )(q, ind3)
    tspec = pl.BlockSpec((1, 128), lambda i: (0, 0))
    return pl.pallas_call(
        _mol_apply_body, grid=grid,
        in_specs=[nspec, ispec, sspec, tspec], out_specs=nspec,
        out_shape=jax.ShapeDtypeStruct((NPAD, 1), _f32),
    )(q, ind3, st, tq)


def kernel(Z, R, e_source, e_target, molecule_ind, total_charge, params):
    Z = Z.astype(_i32)
    esrc = e_source.astype(_i32)
    etgt = e_target.astype(_i32)
    z_pad = jnp.concatenate([Z, jnp.zeros((NPAD - N_ATOM,), _i32)])
    h0, q0 = _embed_call(z_pad, params["embed"], params["guess"])
    geom = _geom_call(R.astype(_f32), esrc, etgt)
    S = jnp.concatenate(
        [_edge_call(True, 0, SW // 2, h0, geom, esrc, etgt),
         _edge_call(True, SW // 2, SW // 2, h0, geom, esrc, etgt)], axis=2)

    charge = q0.reshape(NPAD, 1)
    dipole = jnp.zeros((NPAD, 3), _f32)
    qpole = jnp.zeros((NPAD, 9), _f32)
    h = h0
    for i in range(N_MSG):
        T = _edge_call(False, 0, DW, h, geom, esrc, etgt)
        ps = params
        w1 = jnp.concatenate(
            [ps[n % i][0][0] for n in ("charge_update_%d", "charge_readout_%d",
                                       "dipole_update_%d", "qpole1_update_%d",
                                       "qpole2_update_%d")], axis=1)
        b1 = jnp.concatenate(
            [ps[n % i][0][1] for n in ("charge_update_%d", "charge_readout_%d",
                                       "dipole_update_%d", "qpole1_update_%d",
                                       "qpole2_update_%d")]).reshape(1, -1)
        names = ("charge_update_%d", "charge_readout_%d", "dipole_update_%d",
                 "qpole1_update_%d", "qpole2_update_%d")
        w2s = [ps[n % i][1][0] for n in names]
        b2 = jnp.concatenate([ps[n % i][1][1] for n in names]).reshape(1, -1)
        w3s = [ps[n % i][2][0] for n in names]
        b3 = jnp.concatenate([ps[n % i][2][1] for n in names]).reshape(1, -1)
        wh, bh = ps["charge_update_%d" % i][3]
        wc, bc = ps["charge_readout_%d" % i][3]
        wd, bd = ps["dipole_readout_%d" % i]
        wq, bq = ps["qpole_readout_%d" % i]
        weights = ([w1, b1] + w2s + [b2] + w3s
                   + [b3, wh, bh.reshape(1, -1), wc, bc.reshape(1, -1),
                      wd, bd.reshape(1, -1), wq, bq.reshape(1, -1)])
        h, charge, dipole, qpole = _ffn_call(
            h0, h, S[0], S[1], T[0], T[1], charge, dipole, qpole, weights)

    ind3 = jnp.concatenate(
        [molecule_ind.astype(_i32),
         jnp.full((NPAD - N_ATOM,), 127, _i32)]).reshape(NPAD // BN, 1, BN)
    tq = jnp.concatenate([total_charge.astype(_f32),
                          jnp.zeros((128 - N_MOL,), _f32)]).reshape(1, 128)
    charge = _mol_call(charge, ind3, tq)

    return (charge[:N_ATOM],
            dipole[:N_ATOM],
            qpole[:N_ATOM].reshape(N_ATOM, 3, 3))
